# baseline (speedup 1.0000x reference)
import functools

import jax
import jax.numpy as jnp
from jax import lax
from jax.experimental import pallas as pl
from jax.experimental.pallas import tpu as pltpu
from jax.experimental.pallas import tpu_sc as plsc

B = 4096
D = 64
NC = 2
NS = 16
NW = NC * NS
IPW = B // NW

BIG_KS = (8, 5, 40, 5, 40, 5, 40, 40)
BIG_CHUNK_ITEMS = (64, 128, 16, 128, 16, 128, 16, 16)

TB = 512
SMALL_NS = (4, 21, 21, 21, 21)
POOLED_NS = (21, 21, 21, 21, 21, 21, 13, 21, 4)


def _make_sc_gather_body(ks, chunk_items, bufn):
    nf = len(ks)

    def body(*refs):
        tables = refs[0:nf]
        idxs = refs[nf:2 * nf]
        outs = refs[2 * nf:3 * nf]
        (idx_a, idx_b, rows_a, rows_b, pool_v, sem_a, sem_b) = refs[3 * nf:]
        bufs = ((idx_a, rows_a, sem_a), (idx_b, rows_b, sem_b))

        c = lax.axis_index("c")
        s = lax.axis_index("s")
        wid = s * NC + c

        def fire(f, n, items, k, buf, it_base):
            idx_v, rows_v, sem = buf
            pltpu.sync_copy(idxs[f].at[pl.ds(it_base * k, n)],
                            idx_v.at[pl.ds(0, n)])
            pltpu.async_copy(tables[f].at[idx_v.at[pl.ds(0, n)]],
                             rows_v.at[pl.ds(0, n)], sem)

        def wait(f, n, buf):
            _, rows_v, sem = buf
            pltpu.make_async_copy(tables[f].at[pl.ds(0, n)],
                                  rows_v.at[pl.ds(0, n)], sem).wait()

        def pool_and_out(f, k, items, inv, buf, it_base):
            _, rows_v, _ = buf

            def item_body(i, _):
                for q in range(4):
                    acc = rows_v[i * k, pl.ds(q * 16, 16)]
                    for j in range(1, k):
                        acc = acc + rows_v[i * k + j, pl.ds(q * 16, 16)]
                    pool_v[i, pl.ds(q * 16, 16)] = acc * inv
                return 0

            lax.fori_loop(0, items, item_body, 0)
            pltpu.sync_copy(pool_v.at[pl.ds(0, items)],
                            outs[f].at[pl.ds(it_base, items)])

        for f in range(nf):
            k = ks[f]
            items = chunk_items[f]
            n = items * k
            inv = 1.0 / k
            nch = IPW // items
            if f == 0:
                fire(0, n, items, k, bufs[0], wid * IPW)
            if nch == 1:
                if f + 1 < nf:
                    k2 = ks[f + 1]
                    fire(f + 1, chunk_items[f + 1] * k2, chunk_items[f + 1],
                         k2, bufs[(f + 1) % 2], wid * IPW)
                wait(f, n, bufs[f % 2])
                pool_and_out(f, k, items, inv, bufs[f % 2], wid * IPW)
                continue

            assert nch % 2 == 0

            def dpair(i, _, f=f, k=k, items=items, n=n, inv=inv, nch=nch):
                base = wid * IPW
                ch = 2 * i
                fire(f, n, items, k, bufs[1], base + (ch + 1) * items)
                wait(f, n, bufs[0])
                pool_and_out(f, k, items, inv, bufs[0], base + ch * items)

                @pl.when(ch + 2 < nch)
                def _():
                    fire(f, n, items, k, bufs[0], base + (ch + 2) * items)

                wait(f, n, bufs[1])
                pool_and_out(f, k, items, inv, bufs[1], base + (ch + 1) * items)
                return 0

            lax.fori_loop(0, nch // 2, dpair, 0)
            if f + 1 < nf:
                k2 = ks[f + 1]
                n2 = chunk_items[f + 1] * k2
                fire(f + 1, n2, chunk_items[f + 1], k2,
                     bufs[(f + 1) % 2 if (IPW // chunk_items[f + 1]) == 1 else 0],
                     wid * IPW)

    return body


K_URI = 5
URI_ITEMS = 64
URI_N = URI_ITEMS * K_URI


def _sc_uri_body(refs):
    (cat_ta, artist, idx_t, idx_al, idx_ar,
     out_t, out_ar, out_al, idx_v, rows2_v, rowsa_v, pool_v, sem) = refs

    c = lax.axis_index("c")
    s = lax.axis_index("s")
    wid = s * NC + c
    inv = 1.0 / K_URI

    def pool(rows_v, qoff, it_base, out):
        def item_body(i, _):
            for q in range(4):
                acc = rows_v[i * K_URI, pl.ds((qoff + q) * 16, 16)]
                for j in range(1, K_URI):
                    acc = acc + rows_v[i * K_URI + j,
                                       pl.ds((qoff + q) * 16, 16)]
                pool_v[i, pl.ds(q * 16, 16)] = acc * inv
            return 0

        lax.fori_loop(0, URI_ITEMS, item_body, 0)
        pltpu.sync_copy(pool_v.at[pl.ds(0, URI_ITEMS)],
                        out.at[pl.ds(it_base, URI_ITEMS)])

    def stream_pass(idx_hbm, qoff, out):
        def chunk(ch, _):
            it_base = wid * IPW + ch * URI_ITEMS
            pltpu.sync_copy(idx_hbm.at[pl.ds(it_base * K_URI, URI_N)], idx_v)
            pltpu.async_copy(cat_ta.at[idx_v], rows2_v, sem).wait()
            pool(rows2_v, qoff, it_base, out)
            return 0

        lax.fori_loop(0, IPW // URI_ITEMS, chunk, 0)

    stream_pass(idx_t, 0, out_t)
    stream_pass(idx_al, 4, out_al)

    def art_chunk(ch, _):
        it_base = wid * IPW + ch * URI_ITEMS
        pltpu.sync_copy(idx_ar.at[pl.ds(it_base * K_URI, URI_N)], idx_v)

        def issue(g, _):
            vec = idx_v[pl.ds(g * 16, 16)]
            for j in range(16):
                r = vec[j]
                pltpu.async_copy(artist.at[pl.ds(r, 1), :],
                                 rowsa_v.at[pl.ds(g * 16 + j, 1), :], sem)
            return 0

        lax.fori_loop(0, URI_N // 16, issue, 0)
        pltpu.make_async_copy(artist.at[pl.ds(0, URI_N), :],
                              rowsa_v, sem).wait()
        pool(rowsa_v, 0, it_base, out_ar)
        return 0

    lax.fori_loop(0, IPW // URI_ITEMS, art_chunk, 0)


def _sc_gather_uri(cat_ta, artist, idx_t, idx_al, idx_ar):
    mesh = plsc.VectorSubcoreMesh(core_axis_name="c", subcore_axis_name="s",
                                  num_cores=NC, num_subcores=NS)
    scratch = [pltpu.VMEM((URI_N,), jnp.int32),
               pltpu.VMEM((URI_N, 2 * D), jnp.float32),
               pltpu.VMEM((URI_N, D), jnp.float32),
               pltpu.VMEM((IPW, D), jnp.float32),
               pltpu.SemaphoreType.DMA]
    fn = pl.kernel(
        lambda *refs: _sc_uri_body(refs),
        out_type=[jax.ShapeDtypeStruct((B, D), jnp.float32)
                  for _ in range(3)],
        mesh=mesh,
        scratch_types=scratch,
        compiler_params=pltpu.CompilerParams(use_tc_tiling_on_sc=True),
    )
    return fn(cat_ta, artist, idx_t, idx_al, idx_ar)


def _sc_gather(tables, idxs, ks, chunk_items):
    bufn = max(it * k for it, k in zip(chunk_items, ks))
    mesh = plsc.VectorSubcoreMesh(core_axis_name="c", subcore_axis_name="s",
                                  num_cores=NC, num_subcores=NS)
    scratch = [pltpu.VMEM((bufn,), jnp.int32), pltpu.VMEM((bufn,), jnp.int32),
               pltpu.VMEM((bufn, D), jnp.float32),
               pltpu.VMEM((bufn, D), jnp.float32),
               pltpu.VMEM((IPW, D), jnp.float32),
               pltpu.SemaphoreType.DMA, pltpu.SemaphoreType.DMA]
    fn = pl.kernel(
        _make_sc_gather_body(ks, chunk_items, bufn),
        out_type=[jax.ShapeDtypeStruct((B, D), jnp.float32)
                  for _ in range(len(ks))],
        mesh=mesh,
        scratch_types=scratch,
        compiler_params=pltpu.CompilerParams(use_tc_tiling_on_sc=False),
    )
    return fn(*tables, *idxs)


def _tc_dense_body(bf0, bf1, bf2, bf3, bf4, bf5, bf6, bf7, sidx,
                   t_collab, t_dur, t_songs, t_artists, t_albums,
                   t_dursongs, t_pop, t_apop, t_fol, t_dance, t_energy,
                   t_key, t_loud, t_mode, w1, b1, w2, b2, out):
    cols = sidx[...]

    def onehot(col, n):
        c = cols[:, col][:, None]
        i = lax.broadcasted_iota(jnp.int32, (TB, n), 1)
        return (c == i).astype(jnp.float32)

    def np_feat(col, table):
        n = table.shape[0]
        return jnp.dot(onehot(col, n), table[...],
                       preferred_element_type=jnp.float32)

    def pooled_feat(col0, table):
        n = table.shape[0]
        h = onehot(col0, n)
        for j in range(1, 5):
            h = h + onehot(col0 + j, n)
        return jnp.dot(h, table[...], preferred_element_type=jnp.float32) * 0.2

    feats = [
        bf0[...],
        np_feat(0, t_collab), np_feat(1, t_dur), np_feat(2, t_songs),
        np_feat(3, t_artists), np_feat(4, t_albums),
        bf1[...], bf2[...], bf3[...], bf4[...], bf5[...],
        bf6[...], bf7[...],
        pooled_feat(5, t_dursongs), pooled_feat(10, t_pop),
        pooled_feat(15, t_apop), pooled_feat(20, t_fol),
        pooled_feat(25, t_dance), pooled_feat(30, t_energy),
        pooled_feat(35, t_key), pooled_feat(40, t_loud),
        pooled_feat(45, t_mode),
    ]
    x = jnp.concatenate(feats, axis=1)
    h = jnp.dot(x, w1[...], preferred_element_type=jnp.float32) + b1[...]
    h = jnp.maximum(h, 0.0)
    out[...] = jnp.dot(h, w2[...], preferred_element_type=jnp.float32) + b2[...]


def _tc_dense(big_feats, sidx, small_tables, w1, b1, w2, b2):
    grid = (B // TB,)
    bf_spec = pl.BlockSpec((TB, D), lambda i: (i, 0))
    full = lambda arr: pl.BlockSpec(arr.shape, lambda i: (0,) * arr.ndim)
    in_specs = ([bf_spec] * 8
                + [pl.BlockSpec((TB, 50), lambda i: (i, 0))]
                + [full(t) for t in small_tables]
                + [full(w1), full(b1), full(w2), full(b2)])
    return pl.pallas_call(
        _tc_dense_body,
        grid=grid,
        in_specs=in_specs,
        out_specs=pl.BlockSpec((TB, 128), lambda i: (i, 0)),
        out_shape=jax.ShapeDtypeStruct((B, 128), jnp.float32),
    )(*big_feats, sidx, *small_tables, w1, b1, w2, b2)


def kernel(pl_name_src_tokens, pl_collaborative, pl_duration_bucket,
           num_pl_songs_bucket, num_pl_artists_bucket, num_pl_albums_bucket,
           track_uri_pl, track_name_pl_tokens, artist_uri_pl,
           artist_name_pl_tokens, album_uri_pl, album_name_pl_tokens,
           artist_genres_pl_tokens, duration_ms_songs_pl_bucket,
           track_pop_pl_bucket, artist_pop_pl_bucket,
           artists_followers_pl_bucket, track_danceability_pl_bucket,
           track_energy_pl_bucket, track_key_pl, track_loudness_pl_bucket,
           track_mode_pl, emb_pl_name_src, emb_pl_collab, emb_pl_duration,
           emb_num_songs, emb_num_artists, emb_num_albums, emb_track_uri,
           emb_track_name, emb_artist_uri, emb_artist_name, emb_album_uri,
           emb_album_name, emb_artist_genres, emb_dur_songs, emb_track_pop,
           emb_artist_pop, emb_followers, emb_dance, emb_energy, emb_key,
           emb_loudness, emb_mode, W1, b1, W2, b2):
    tok_feats = _sc_gather(
        [emb_pl_name_src, emb_track_name, emb_artist_name, emb_album_name,
         emb_artist_genres],
        [pl_name_src_tokens.reshape(-1), track_name_pl_tokens.reshape(-1),
         artist_name_pl_tokens.reshape(-1), album_name_pl_tokens.reshape(-1),
         artist_genres_pl_tokens.reshape(-1)],
        ks=(8, 40, 40, 40, 40), chunk_items=(64, 16, 16, 16, 16))
    cat_ta = jnp.concatenate([emb_track_uri, emb_album_uri], axis=1)
    uri_feats = _sc_gather_uri(
        cat_ta, emb_artist_uri,
        track_uri_pl.reshape(-1), album_uri_pl.reshape(-1),
        artist_uri_pl.reshape(-1))

    sidx = jnp.concatenate(
        [pl_collaborative[:, None], pl_duration_bucket[:, None],
         num_pl_songs_bucket[:, None], num_pl_artists_bucket[:, None],
         num_pl_albums_bucket[:, None], duration_ms_songs_pl_bucket,
         track_pop_pl_bucket, artist_pop_pl_bucket,
         artists_followers_pl_bucket, track_danceability_pl_bucket,
         track_energy_pl_bucket, track_key_pl, track_loudness_pl_bucket,
         track_mode_pl], axis=1)
    small_tables = [emb_pl_collab, emb_pl_duration, emb_num_songs,
                    emb_num_artists, emb_num_albums, emb_dur_songs,
                    emb_track_pop, emb_artist_pop, emb_followers, emb_dance,
                    emb_energy, emb_key, emb_loudness, emb_mode]
    big_feats = [tok_feats[0], uri_feats[0], tok_feats[1], uri_feats[1],
                 tok_feats[2], uri_feats[2], tok_feats[3], tok_feats[4]]
    return _tc_dense(big_feats, sidx, small_tables,
                     W1, b1[None, :], W2, b2[None, :])


# speedup vs baseline: 1.0360x; 1.0360x over previous
import functools

import jax
import jax.numpy as jnp
from jax import lax
from jax.experimental import pallas as pl
from jax.experimental.pallas import tpu as pltpu
from jax.experimental.pallas import tpu_sc as plsc

B = 4096
D = 64
NC = 2
NS = 16
NW = NC * NS
IPW = B // NW

BIG_KS = (8, 5, 40, 5, 40, 5, 40, 40)
BIG_CHUNK_ITEMS = (64, 128, 16, 128, 16, 128, 16, 16)

TB = 512
SMALL_NS = (4, 21, 21, 21, 21)
POOLED_NS = (21, 21, 21, 21, 21, 21, 13, 21, 4)


def _make_sc_gather_body(ks, chunk_items, bufn):
    nf = len(ks)

    def body(*refs):
        tables = refs[0:nf]
        idxs = refs[nf:2 * nf]
        outs = refs[2 * nf:3 * nf]
        (idx_a, idx_b, rows_a, rows_b, pool_v, sem_a, sem_b) = refs[3 * nf:]
        bufs = ((idx_a, rows_a, sem_a), (idx_b, rows_b, sem_b))

        c = lax.axis_index("c")
        s = lax.axis_index("s")
        wid = s * NC + c

        def fire(f, n, items, k, buf, it_base):
            idx_v, rows_v, sem = buf
            pltpu.sync_copy(idxs[f].at[pl.ds(it_base * k, n)],
                            idx_v.at[pl.ds(0, n)])
            pltpu.async_copy(tables[f].at[idx_v.at[pl.ds(0, n)]],
                             rows_v.at[pl.ds(0, n)], sem)

        def wait(f, n, buf):
            _, rows_v, sem = buf
            pltpu.make_async_copy(tables[f].at[pl.ds(0, n)],
                                  rows_v.at[pl.ds(0, n)], sem).wait()

        def pool_and_out(f, k, items, inv, buf, it_base):
            _, rows_v, _ = buf

            def item_body(i, _):
                for q in range(4):
                    acc = rows_v[i * k, pl.ds(q * 16, 16)]
                    for j in range(1, k):
                        acc = acc + rows_v[i * k + j, pl.ds(q * 16, 16)]
                    pool_v[i, pl.ds(q * 16, 16)] = acc * inv
                return 0

            lax.fori_loop(0, items, item_body, 0)
            pltpu.sync_copy(pool_v.at[pl.ds(0, items)],
                            outs[f].at[pl.ds(it_base, items)])

        for f in range(nf):
            k = ks[f]
            items = chunk_items[f]
            n = items * k
            inv = 1.0 / k
            nch = IPW // items
            if f == 0:
                fire(0, n, items, k, bufs[0], wid * IPW)
            if nch == 1:
                if f + 1 < nf:
                    k2 = ks[f + 1]
                    fire(f + 1, chunk_items[f + 1] * k2, chunk_items[f + 1],
                         k2, bufs[(f + 1) % 2], wid * IPW)
                wait(f, n, bufs[f % 2])
                pool_and_out(f, k, items, inv, bufs[f % 2], wid * IPW)
                continue

            assert nch % 2 == 0

            def dpair(i, _, f=f, k=k, items=items, n=n, inv=inv, nch=nch):
                base = wid * IPW
                ch = 2 * i
                fire(f, n, items, k, bufs[1], base + (ch + 1) * items)
                wait(f, n, bufs[0])
                pool_and_out(f, k, items, inv, bufs[0], base + ch * items)

                @pl.when(ch + 2 < nch)
                def _():
                    fire(f, n, items, k, bufs[0], base + (ch + 2) * items)

                wait(f, n, bufs[1])
                pool_and_out(f, k, items, inv, bufs[1], base + (ch + 1) * items)
                return 0

            lax.fori_loop(0, nch // 2, dpair, 0)
            if f + 1 < nf:
                k2 = ks[f + 1]
                n2 = chunk_items[f + 1] * k2
                fire(f + 1, n2, chunk_items[f + 1], k2,
                     bufs[(f + 1) % 2 if (IPW // chunk_items[f + 1]) == 1 else 0],
                     wid * IPW)

    return body


def _sc_rowdma_body(nf, k, items):
    n = items * k

    def body(*refs):
        tables = refs[0:nf]
        idxs = refs[nf:2 * nf]
        outs = refs[2 * nf:3 * nf]
        idx_v, rows_v, pool_v, sem = refs[3 * nf:]

        c = lax.axis_index("c")
        s = lax.axis_index("s")
        wid = s * NC + c
        base = wid * IPW

        for f in range(nf):
            pltpu.sync_copy(idxs[f].at[pl.ds(base * k, n)], idx_v)

            def issue(g, _, f=f):
                vec = idx_v[pl.ds(g * 16, 16)]
                for j in range(16):
                    r = vec[j]
                    pltpu.async_copy(
                        tables[f].at[pl.ds(r, 1), :],
                        rows_v.at[pl.ds(g * 16 + j, 1), :], sem)
                return 0

            lax.fori_loop(0, n // 16, issue, 0)
            pltpu.make_async_copy(tables[f].at[pl.ds(0, n), :],
                                  rows_v, sem).wait()

            inv = 1.0 / k

            def item_body(i, _):
                for q in range(4):
                    acc = rows_v[i * k, pl.ds(q * 16, 16)]
                    for j in range(1, k):
                        acc = acc + rows_v[i * k + j, pl.ds(q * 16, 16)]
                    pool_v[i, pl.ds(q * 16, 16)] = acc * inv
                return 0

            lax.fori_loop(0, items, item_body, 0)
            pltpu.sync_copy(pool_v.at[pl.ds(0, items)],
                            outs[f].at[pl.ds(base, items)])

    return body


def _sc_gather_rowdma(tables, idxs, k, items):
    n = items * k
    mesh = plsc.VectorSubcoreMesh(core_axis_name="c", subcore_axis_name="s",
                                  num_cores=NC, num_subcores=NS)
    scratch = [pltpu.VMEM((n,), jnp.int32),
               pltpu.VMEM((n, D), jnp.float32),
               pltpu.VMEM((IPW, D), jnp.float32),
               pltpu.SemaphoreType.DMA]
    fn = pl.kernel(
        _sc_rowdma_body(len(tables), k, items),
        out_type=[jax.ShapeDtypeStruct((B, D), jnp.float32)
                  for _ in range(len(tables))],
        mesh=mesh,
        scratch_types=scratch,
        compiler_params=pltpu.CompilerParams(use_tc_tiling_on_sc=True),
    )
    return fn(*tables, *idxs)


def _sc_gather(tables, idxs, ks, chunk_items):
    bufn = max(it * k for it, k in zip(chunk_items, ks))
    mesh = plsc.VectorSubcoreMesh(core_axis_name="c", subcore_axis_name="s",
                                  num_cores=NC, num_subcores=NS)
    scratch = [pltpu.VMEM((bufn,), jnp.int32), pltpu.VMEM((bufn,), jnp.int32),
               pltpu.VMEM((bufn, D), jnp.float32),
               pltpu.VMEM((bufn, D), jnp.float32),
               pltpu.VMEM((IPW, D), jnp.float32),
               pltpu.SemaphoreType.DMA, pltpu.SemaphoreType.DMA]
    fn = pl.kernel(
        _make_sc_gather_body(ks, chunk_items, bufn),
        out_type=[jax.ShapeDtypeStruct((B, D), jnp.float32)
                  for _ in range(len(ks))],
        mesh=mesh,
        scratch_types=scratch,
        compiler_params=pltpu.CompilerParams(use_tc_tiling_on_sc=False),
    )
    return fn(*tables, *idxs)


def _tc_dense_body(bf0, bf1, bf2, bf3, bf4, bf5, bf6, bf7, sidx,
                   t_collab, t_dur, t_songs, t_artists, t_albums,
                   t_dursongs, t_pop, t_apop, t_fol, t_dance, t_energy,
                   t_key, t_loud, t_mode, w1, b1, w2, b2, out):
    cols = sidx[...]

    def onehot(col, n):
        c = cols[:, col][:, None]
        i = lax.broadcasted_iota(jnp.int32, (TB, n), 1)
        return (c == i).astype(jnp.float32)

    def np_feat(col, table):
        n = table.shape[0]
        return jnp.dot(onehot(col, n), table[...],
                       preferred_element_type=jnp.float32)

    def pooled_feat(col0, table):
        n = table.shape[0]
        h = onehot(col0, n)
        for j in range(1, 5):
            h = h + onehot(col0 + j, n)
        return jnp.dot(h, table[...], preferred_element_type=jnp.float32) * 0.2

    feats = [
        bf0[...],
        np_feat(0, t_collab), np_feat(1, t_dur), np_feat(2, t_songs),
        np_feat(3, t_artists), np_feat(4, t_albums),
        bf1[...], bf2[...], bf3[...], bf4[...], bf5[...],
        bf6[...], bf7[...],
        pooled_feat(5, t_dursongs), pooled_feat(10, t_pop),
        pooled_feat(15, t_apop), pooled_feat(20, t_fol),
        pooled_feat(25, t_dance), pooled_feat(30, t_energy),
        pooled_feat(35, t_key), pooled_feat(40, t_loud),
        pooled_feat(45, t_mode),
    ]
    x = jnp.concatenate(feats, axis=1).astype(jnp.bfloat16)
    h = jnp.dot(x, w1[...], preferred_element_type=jnp.float32) + b1[...]
    h = jnp.maximum(h, 0.0).astype(jnp.bfloat16)
    out[...] = jnp.dot(h, w2[...], preferred_element_type=jnp.float32) + b2[...]


def _tc_dense(big_feats, sidx, small_tables, w1, b1, w2, b2):
    grid = (B // TB,)
    bf_spec = pl.BlockSpec((TB, D), lambda i: (i, 0))
    full = lambda arr: pl.BlockSpec(arr.shape, lambda i: (0,) * arr.ndim)
    in_specs = ([bf_spec] * 8
                + [pl.BlockSpec((TB, 50), lambda i: (i, 0))]
                + [full(t) for t in small_tables]
                + [full(w1), full(b1), full(w2), full(b2)])
    return pl.pallas_call(
        _tc_dense_body,
        grid=grid,
        in_specs=in_specs,
        out_specs=pl.BlockSpec((TB, 128), lambda i: (i, 0)),
        out_shape=jax.ShapeDtypeStruct((B, 128), jnp.float32),
    )(*big_feats, sidx, *small_tables, w1, b1, w2, b2)


def kernel(pl_name_src_tokens, pl_collaborative, pl_duration_bucket,
           num_pl_songs_bucket, num_pl_artists_bucket, num_pl_albums_bucket,
           track_uri_pl, track_name_pl_tokens, artist_uri_pl,
           artist_name_pl_tokens, album_uri_pl, album_name_pl_tokens,
           artist_genres_pl_tokens, duration_ms_songs_pl_bucket,
           track_pop_pl_bucket, artist_pop_pl_bucket,
           artists_followers_pl_bucket, track_danceability_pl_bucket,
           track_energy_pl_bucket, track_key_pl, track_loudness_pl_bucket,
           track_mode_pl, emb_pl_name_src, emb_pl_collab, emb_pl_duration,
           emb_num_songs, emb_num_artists, emb_num_albums, emb_track_uri,
           emb_track_name, emb_artist_uri, emb_artist_name, emb_album_uri,
           emb_album_name, emb_artist_genres, emb_dur_songs, emb_track_pop,
           emb_artist_pop, emb_followers, emb_dance, emb_energy, emb_key,
           emb_loudness, emb_mode, W1, b1, W2, b2):
    tok_feats = _sc_gather(
        [emb_pl_name_src, emb_track_name, emb_artist_name, emb_album_name,
         emb_artist_genres],
        [pl_name_src_tokens.reshape(-1), track_name_pl_tokens.reshape(-1),
         artist_name_pl_tokens.reshape(-1), album_name_pl_tokens.reshape(-1),
         artist_genres_pl_tokens.reshape(-1)],
        ks=(8, 40, 40, 40, 40), chunk_items=(64, 16, 16, 16, 16))
    uri_feats = _sc_gather_rowdma(
        [emb_track_uri, emb_artist_uri, emb_album_uri],
        [track_uri_pl.reshape(-1), artist_uri_pl.reshape(-1),
         album_uri_pl.reshape(-1)],
        k=5, items=IPW)

    sidx = jnp.concatenate(
        [pl_collaborative[:, None], pl_duration_bucket[:, None],
         num_pl_songs_bucket[:, None], num_pl_artists_bucket[:, None],
         num_pl_albums_bucket[:, None], duration_ms_songs_pl_bucket,
         track_pop_pl_bucket, artist_pop_pl_bucket,
         artists_followers_pl_bucket, track_danceability_pl_bucket,
         track_energy_pl_bucket, track_key_pl, track_loudness_pl_bucket,
         track_mode_pl], axis=1)
    small_tables = [emb_pl_collab, emb_pl_duration, emb_num_songs,
                    emb_num_artists, emb_num_albums, emb_dur_songs,
                    emb_track_pop, emb_artist_pop, emb_followers, emb_dance,
                    emb_energy, emb_key, emb_loudness, emb_mode]
    big_feats = [tok_feats[0], uri_feats[0], tok_feats[1], uri_feats[1],
                 tok_feats[2], uri_feats[2], tok_feats[3], tok_feats[4]]
    return _tc_dense(big_feats, sidx, small_tables,
                     W1.astype(jnp.bfloat16), b1[None, :],
                     W2.astype(jnp.bfloat16), b2[None, :])
